# baseline (speedup 1.0000x reference)
import functools

import jax
import jax.numpy as jnp
from jax import lax
from jax.experimental import pallas as pl
from jax.experimental.pallas import tpu as pltpu
from jax.experimental.pallas import tpu_sc as plsc

_NUM_CORES = 2
_NUM_SUBCORES = 16
_NW = _NUM_CORES * _NUM_SUBCORES
_NBUF = 5
_GLEAD = 3


@functools.partial(jax.jit, static_argnums=(2, 3))
def _sc_gather(table, idx, n_total, chunk):
    dim = table.shape[1]
    n_per_w = n_total // _NW
    n_chunks = n_per_w // chunk
    n_head = _NBUF - _GLEAD
    n_tail = _NBUF - _GLEAD
    n_main = n_chunks - n_head - n_tail
    assert n_main % _NBUF == 0 and n_chunks >= 2 * _NBUF
    mesh = plsc.VectorSubcoreMesh(core_axis_name="c", subcore_axis_name="s")

    @functools.partial(
        pl.kernel,
        out_type=jax.ShapeDtypeStruct((n_total, dim), jnp.float32),
        mesh=mesh,
        scratch_types=[
            pltpu.VMEM((n_per_w,), jnp.int32),
            [pltpu.VMEM((chunk, dim), jnp.float32) for _ in range(_NBUF)],
            [pltpu.SemaphoreType.DMA for _ in range(_NBUF)],
            [pltpu.SemaphoreType.DMA for _ in range(_NBUF)],
        ],
    )
    def k(table_hbm, idx_hbm, out_hbm, idx_v, bufs, sem_g, sem_s):
        wid = lax.axis_index("s") * _NUM_CORES + lax.axis_index("c")
        base = wid * n_per_w
        pltpu.sync_copy(idx_hbm.at[pl.ds(base, n_per_w)], idx_v)

        def gather_copy(c, b):
            return pltpu.make_async_copy(
                table_hbm.at[idx_v.at[pl.ds(c * chunk, chunk)]], bufs[b], sem_g[b]
            )

        def scatter_copy(c, b):
            return pltpu.make_async_copy(
                bufs[b], out_hbm.at[pl.ds(base + c * chunk, chunk)], sem_s[b]
            )

        def step(c, j, wait_prev, start_next):
            nb = (j + _GLEAD) % _NBUF
            if wait_prev:
                scatter_copy(c + _GLEAD - _NBUF, nb).wait()
            if start_next is True:
                gather_copy(c + _GLEAD, nb).start()
            elif start_next is not False:

                @pl.when(start_next)
                def _g():
                    gather_copy(c + _GLEAD, nb).start()

            gather_copy(c, j).wait()
            scatter_copy(c, j).start()

        for c in range(_GLEAD):
            gather_copy(c, c).start()

        for c in range(n_head):
            step(c, c, wait_prev=False, start_next=True)

        @pl.loop(n_head, n_head + n_main, step=_NBUF)
        def _block(o):
            for j0 in range(_NBUF):
                c = o + j0
                step(c, (n_head + j0) % _NBUF, True, c + _GLEAD < n_chunks)

        for c in range(n_head + n_main, n_chunks):
            step(c, c % _NBUF, wait_prev=True, start_next=False)
        for c in range(n_chunks - _NBUF + _GLEAD, n_chunks):
            scatter_copy(c, c % _NBUF).wait()

    return k(table, idx)


def kernel(Position, pos_embed_weight):
    b, s = Position.shape
    idx = Position.reshape(-1)
    n_sc = 12544
    out_sc = _sc_gather(pos_embed_weight, idx[:n_sc], n_sc, 8)
    out_tc = jnp.take(pos_embed_weight, idx[n_sc:], axis=0)
    out = jnp.concatenate([out_sc, out_tc], axis=0)
    return out.reshape(b, s, pos_embed_weight.shape[1])


# speedup vs baseline: 1.9846x; 1.9846x over previous
import functools

import jax
import jax.numpy as jnp
from jax import lax
from jax.experimental import pallas as pl
from jax.experimental.pallas import tpu as pltpu
from jax.experimental.pallas import tpu_sc as plsc

_NUM_CORES = 2
_NUM_SUBCORES = 16
_NW = _NUM_CORES * _NUM_SUBCORES
_NBUF = 5
_GLEAD = 3


@functools.partial(jax.jit, static_argnums=(2, 3))
def _sc_gather(table, idx, n_total, chunk):
    dim = table.shape[1]
    n_per_w = n_total // _NW
    s_len = idx.shape[1]
    w_per_row = s_len // n_per_w
    n_chunks = n_per_w // chunk
    n_head = _NBUF - _GLEAD
    n_tail = _NBUF - _GLEAD
    n_main = n_chunks - n_head - n_tail
    assert n_main % _NBUF == 0 and n_chunks >= 2 * _NBUF
    mesh = plsc.VectorSubcoreMesh(core_axis_name="c", subcore_axis_name="s")

    @functools.partial(
        pl.kernel,
        out_type=jax.ShapeDtypeStruct((n_total, dim), jnp.float32),
        mesh=mesh,
        scratch_types=[
            pltpu.VMEM((n_per_w,), jnp.int32),
            [pltpu.VMEM((chunk, dim), jnp.float32) for _ in range(_NBUF)],
            [pltpu.SemaphoreType.DMA for _ in range(_NBUF)],
            [pltpu.SemaphoreType.DMA for _ in range(_NBUF)],
        ],
    )
    def k(table_hbm, idx_hbm, out_hbm, idx_v, bufs, sem_g, sem_s):
        wid = lax.axis_index("c") * _NUM_SUBCORES + lax.axis_index("s")
        base = wid * n_per_w
        pltpu.sync_copy(
            idx_hbm.at[wid // w_per_row, pl.ds((wid % w_per_row) * n_per_w, n_per_w)],
            idx_v,
        )

        def gather_copy(c, b):
            return pltpu.make_async_copy(
                table_hbm.at[idx_v.at[pl.ds(c * chunk, chunk)]], bufs[b], sem_g[b]
            )

        def scatter_copy(c, b):
            return pltpu.make_async_copy(
                bufs[b], out_hbm.at[pl.ds(base + c * chunk, chunk)], sem_s[b]
            )

        def step(c, j, wait_prev, start_next):
            nb = (j + _GLEAD) % _NBUF
            if wait_prev:
                scatter_copy(c + _GLEAD - _NBUF, nb).wait()
            if start_next is True:
                gather_copy(c + _GLEAD, nb).start()
            elif start_next is not False:

                @pl.when(start_next)
                def _g():
                    gather_copy(c + _GLEAD, nb).start()

            gather_copy(c, j).wait()
            scatter_copy(c, j).start()

        for c in range(_GLEAD):
            gather_copy(c, c).start()

        for c in range(n_head):
            step(c, c, wait_prev=False, start_next=True)

        @pl.loop(n_head, n_head + n_main, step=_NBUF)
        def _block(o):
            for j0 in range(_NBUF):
                c = o + j0
                step(c, (n_head + j0) % _NBUF, True, c + _GLEAD < n_chunks)

        for c in range(n_head + n_main, n_chunks):
            step(c, c % _NBUF, wait_prev=True, start_next=False)
        for c in range(n_chunks - _NBUF + _GLEAD, n_chunks):
            scatter_copy(c, c % _NBUF).wait()

    return k(table, idx)


def kernel(Position, pos_embed_weight):
    b, s = Position.shape
    out = _sc_gather(pos_embed_weight, Position, b * s, 8)
    return out.reshape(b, s, pos_embed_weight.shape[1])
